# baseline (speedup 1.0000x reference)
import functools

import jax
import jax.numpy as jnp
from jax import lax
from jax.experimental import pallas as pl
from jax.experimental.pallas import tpu as pltpu
from jax.experimental.pallas import tpu_sc as plsc

HIDDEN = 1024
FFN = 512
NUM_EXPERTS = 8
TOP_K = 2
T = 2048
P = T * TOP_K
M = 256
NB = 24
NR = NB * M
CHUNK = 512


def _route_body(x_ref, rw_ref, tw0_ref, tw1_ref, dk0_ref, dk1_ref,
                bexp_ref, nv_ref, oh0_s, oh1_s, r0_s, r1_s, c0_s, c1_s):
    c = pl.program_id(0)
    x = x_ref[...]
    logits = jnp.dot(x, rw_ref[...], preferred_element_type=jnp.float32)
    probs = jax.nn.softmax(logits, axis=-1)
    lane = jax.lax.broadcasted_iota(jnp.int32, probs.shape, 1)
    m1 = jnp.max(probs, axis=-1, keepdims=True)
    i1 = jnp.min(jnp.where(probs == m1, lane, NUM_EXPERTS), axis=-1,
                 keepdims=True)
    oh0 = lane == i1
    masked = jnp.where(oh0, -jnp.inf, probs)
    m2 = jnp.max(masked, axis=-1, keepdims=True)
    i2 = jnp.min(jnp.where(masked == m2, lane, NUM_EXPERTS), axis=-1,
                 keepdims=True)
    oh1 = lane == i2
    s = m1 + m2
    ones_lg = jnp.ones((1, 128), jnp.float32)
    tw0_ref[...] = (m1 / s) * ones_lg
    tw1_ref[...] = (m2 / s) * ones_lg
    oh0f = oh0.astype(jnp.float32)
    oh1f = oh1.astype(jnp.float32)
    oh0_s[pl.ds(c * CHUNK, CHUNK), :] = oh0f
    oh1_s[pl.ds(c * CHUNK, CHUNK), :] = oh1f

    @pl.when(c == 0)
    def _init():
        c0_s[...] = jnp.zeros((1, NUM_EXPERTS), jnp.float32)
        c1_s[...] = jnp.zeros((1, NUM_EXPERTS), jnp.float32)

    tri = (jax.lax.broadcasted_iota(jnp.int32, (CHUNK, CHUNK), 0)
           > jax.lax.broadcasted_iota(jnp.int32, (CHUNK, CHUNK), 1)
           ).astype(jnp.float32)
    r0_s[pl.ds(c * CHUNK, CHUNK), :] = (
        jnp.dot(tri, oh0f, preferred_element_type=jnp.float32) + c0_s[...])
    r1_s[pl.ds(c * CHUNK, CHUNK), :] = (
        jnp.dot(tri, oh1f, preferred_element_type=jnp.float32) + c1_s[...])
    c0_s[...] += jnp.sum(oh0f, axis=0, keepdims=True)
    c1_s[...] += jnp.sum(oh1f, axis=0, keepdims=True)

    @pl.when(c == T // CHUNK - 1)
    def _epilogue():
        cnt0 = c0_s[...]
        cnt = cnt0 + c1_s[...]
        nb = jnp.floor((cnt + (M - 1)) * (1.0 / M))
        pc = nb * M
        e_iota0 = jax.lax.broadcasted_iota(
            jnp.int32, (NUM_EXPERTS, NUM_EXPERTS), 0)
        e_iota1 = jax.lax.broadcasted_iota(
            jnp.int32, (NUM_EXPERTS, NUM_EXPERTS), 1)
        excl = (e_iota0 < e_iota1).astype(jnp.float32)
        incl = (e_iota0 <= e_iota1).astype(jnp.float32)
        padoff = jnp.dot(pc, excl, preferred_element_type=jnp.float32)
        cumnb = jnp.dot(nb, incl, preferred_element_type=jnp.float32)

        d0 = jnp.sum((padoff + r0_s[...]) * oh0_s[...], axis=-1,
                     keepdims=True)
        d1 = jnp.sum((padoff + cnt0 + r1_s[...]) * oh1_s[...], axis=-1,
                     keepdims=True)
        dk0_ref[...] = d0.astype(jnp.int32)
        dk1_ref[...] = d1.astype(jnp.int32)

        b_iota = jax.lax.broadcasted_iota(
            jnp.int32, (1, NB), 1).astype(jnp.float32)
        b_iota = jnp.minimum(b_iota, cumnb[0, NUM_EXPERTS - 1] - 1.0)
        bexp = jnp.zeros((1, NB), jnp.float32)
        for e in range(NUM_EXPERTS):
            bexp = bexp + jnp.where(b_iota >= cumnb[0, e], 1.0, 0.0)
        bexp_ref[...] = jnp.minimum(bexp, NUM_EXPERTS - 1).astype(jnp.int32)
        nv_ref[...] = cumnb[0:1, NUM_EXPERTS - 1:NUM_EXPERTS].astype(jnp.int32)


def _route(x, rw):
    return pl.pallas_call(
        _route_body,
        grid=(T // CHUNK,),
        in_specs=[
            pl.BlockSpec((CHUNK, HIDDEN), lambda c: (c, 0)),
            pl.BlockSpec((HIDDEN, NUM_EXPERTS), lambda c: (0, 0)),
        ],
        out_specs=[
            pl.BlockSpec((CHUNK, 128), lambda c: (c, 0)),
            pl.BlockSpec((CHUNK, 128), lambda c: (c, 0)),
            pl.BlockSpec((T, 1), lambda c: (0, 0)),
            pl.BlockSpec((T, 1), lambda c: (0, 0)),
            pl.BlockSpec((1, NB), lambda c: (0, 0)),
            pl.BlockSpec((1, 1), lambda c: (0, 0)),
        ],
        out_shape=[
            jax.ShapeDtypeStruct((T, 128), jnp.float32),
            jax.ShapeDtypeStruct((T, 128), jnp.float32),
            jax.ShapeDtypeStruct((T, 1), jnp.int32),
            jax.ShapeDtypeStruct((T, 1), jnp.int32),
            jax.ShapeDtypeStruct((1, NB), jnp.int32),
            jax.ShapeDtypeStruct((1, 1), jnp.int32),
        ],
        scratch_shapes=[pltpu.VMEM((T, NUM_EXPERTS), jnp.float32)] * 4
        + [pltpu.VMEM((1, NUM_EXPERTS), jnp.float32)] * 2,
    )(x, rw)


def _gmm_body(bexp_ref, nv_ref, xs_ref, ws_ref, g_ref, u_ref, d_ref, ys_ref):
    b = pl.program_id(0)

    @pl.when(b < nv_ref[0])
    def _():
        xb = xs_ref[...].astype(jnp.bfloat16)
        a = jnp.dot(xb, g_ref[0].astype(jnp.bfloat16),
                    preferred_element_type=jnp.float32)
        u = jnp.dot(xb, u_ref[0].astype(jnp.bfloat16),
                    preferred_element_type=jnp.float32)
        w = ws_ref[...][:, 0:1]
        h = (jax.nn.silu(a) * u * w).astype(jnp.bfloat16)
        ys_ref[...] = jnp.dot(h, d_ref[0].astype(jnp.bfloat16),
                              preferred_element_type=jnp.float32)


def _gmm(bexp, nv, xs, ws, gw, uw, dw):
    grid_spec = pltpu.PrefetchScalarGridSpec(
        num_scalar_prefetch=2,
        grid=(NB,),
        in_specs=[
            pl.BlockSpec((M, HIDDEN),
                         lambda b, be, nv: (jnp.where(b < nv[0], b, 0), 0)),
            pl.BlockSpec((M, 128),
                         lambda b, be, nv: (jnp.where(b < nv[0], b, 0), 0)),
            pl.BlockSpec((1, HIDDEN, FFN),
                         lambda b, be, nv: (be[b], 0, 0)),
            pl.BlockSpec((1, HIDDEN, FFN),
                         lambda b, be, nv: (be[b], 0, 0)),
            pl.BlockSpec((1, FFN, HIDDEN),
                         lambda b, be, nv: (be[b], 0, 0)),
        ],
        out_specs=pl.BlockSpec(
            (M, HIDDEN), lambda b, be, nv: (jnp.where(b < nv[0], b, NB), 0)),
    )
    return pl.pallas_call(
        _gmm_body,
        grid_spec=grid_spec,
        out_shape=jax.ShapeDtypeStruct((NR + M, HIDDEN), jnp.float32),
    )(bexp, nv, xs, ws, gw, uw, dw)


NW = 32
TOK_W = T // NW
SCCH = 32
LG = 16


@functools.cache
def _sc_kernels():
    mesh = plsc.VectorSubcoreMesh(core_axis_name="c", subcore_axis_name="s",
                                  num_cores=2, num_subcores=16)

    @functools.partial(
        pl.kernel,
        out_type=[
            jax.ShapeDtypeStruct((NR, HIDDEN), jnp.float32),
            jax.ShapeDtypeStruct((NR, 128), jnp.float32),
        ],
        mesh=mesh,
        scratch_types=[
            pltpu.VMEM((TOK_W, HIDDEN), jnp.float32),
            pltpu.VMEM((TOK_W,), jnp.int32),
            pltpu.VMEM((TOK_W,), jnp.int32),
            pltpu.VMEM((TOK_W, 128), jnp.float32),
            pltpu.VMEM((TOK_W, 128), jnp.float32),
            pltpu.SemaphoreType.DMA,
            pltpu.SemaphoreType.DMA,
            pltpu.SemaphoreType.DMA,
            pltpu.SemaphoreType.DMA,
            pltpu.SemaphoreType.DMA,
        ],
    )
    def _scatter_k(x_hbm, dk0_hbm, dk1_hbm, tw0_hbm, tw1_hbm, xs_hbm, ws_hbm,
                   xv, i0v, i1v, w0v, w1v, s0, s1, s2, s3, s4):
        wid = lax.axis_index("s") * 2 + lax.axis_index("c")
        base = wid * TOK_W
        cps = [
            pltpu.async_copy(x_hbm.at[pl.ds(base, TOK_W)], xv, s0),
            pltpu.async_copy(dk0_hbm.at[pl.ds(base, TOK_W)], i0v, s1),
            pltpu.async_copy(dk1_hbm.at[pl.ds(base, TOK_W)], i1v, s2),
            pltpu.async_copy(tw0_hbm.at[pl.ds(base, TOK_W)], w0v, s3),
            pltpu.async_copy(tw1_hbm.at[pl.ds(base, TOK_W)], w1v, s4),
        ]
        for cp in cps:
            cp.wait()
        cps = [
            pltpu.async_copy(xv, xs_hbm.at[i0v], s0),
            pltpu.async_copy(xv, xs_hbm.at[i1v], s1),
            pltpu.async_copy(w0v, ws_hbm.at[i0v], s2),
            pltpu.async_copy(w1v, ws_hbm.at[i1v], s3),
        ]
        for cp in cps:
            cp.wait()

    CCH = 16
    NCH = TOK_W // CCH

    @functools.partial(
        pl.kernel,
        out_type=jax.ShapeDtypeStruct((T, HIDDEN), jnp.float32),
        mesh=mesh,
        scratch_types=[
            pltpu.VMEM((2, CCH, HIDDEN), jnp.float32),
            pltpu.VMEM((2, CCH, HIDDEN), jnp.float32),
            pltpu.VMEM((TOK_W,), jnp.int32),
            pltpu.VMEM((TOK_W,), jnp.int32),
            pltpu.SemaphoreType.DMA,
            pltpu.SemaphoreType.DMA,
            pltpu.SemaphoreType.DMA,
            pltpu.SemaphoreType.DMA,
            pltpu.SemaphoreType.DMA,
        ],
    )
    def _combine_k(ys_hbm, dk0_hbm, dk1_hbm, out_hbm,
                   ya, yb, i0v, i1v, sa0, sa1, sb0, sb1, sw):
        wid = lax.axis_index("s") * 2 + lax.axis_index("c")
        base = wid * TOK_W
        cp0 = pltpu.async_copy(dk0_hbm.at[pl.ds(base, TOK_W)], i0v, sa0)
        cp1 = pltpu.async_copy(dk1_hbm.at[pl.ds(base, TOK_W)], i1v, sb0)
        cp0.wait()
        cp1.wait()
        sa = [sa0, sa1]
        sb = [sb0, sb1]

        def gather(c):
            buf = c % 2
            ca = pltpu.async_copy(
                ys_hbm.at[i0v.at[pl.ds(c * CCH, CCH)]], ya.at[buf], sa[buf])
            cb = pltpu.async_copy(
                ys_hbm.at[i1v.at[pl.ds(c * CCH, CCH)]], yb.at[buf], sb[buf])
            return ca, cb

        pend = gather(0)
        wr = None
        for c in range(NCH):
            buf = c % 2
            ca, cb = pend
            if wr is not None:
                wr.wait()
            if c + 1 < NCH:
                pend = gather(c + 1)
            ca.wait()
            cb.wait()

            def tok(i, _):
                for j in range(HIDDEN // LG):
                    ya[buf, i, pl.ds(j * LG, LG)] = (
                        ya[buf, i, pl.ds(j * LG, LG)]
                        + yb[buf, i, pl.ds(j * LG, LG)])
                return 0

            lax.fori_loop(0, CCH, tok, 0)
            wr = pltpu.async_copy(
                ya.at[buf], out_hbm.at[pl.ds(base + c * CCH, CCH)], sw)
        wr.wait()

    return _scatter_k, _combine_k


def kernel(hidden_states, router_w, gate_w, up_w, down_w):
    B, S, H = hidden_states.shape
    scatter_k, combine_k = _sc_kernels()
    x = hidden_states.reshape(-1, H)
    tw0, tw1, dk0, dk1, bexp, nv = _route(x, router_w)
    dk0 = dk0.reshape(-1)
    dk1 = dk1.reshape(-1)
    xs, ws = scatter_k(x, dk0, dk1, tw0, tw1)
    ys = _gmm(bexp.reshape(-1), nv.reshape(-1), xs, ws,
              gate_w, up_w, down_w)
    out = combine_k(ys, dk0, dk1)
    return out.reshape(B, S, H)


# speedup vs baseline: 1.0611x; 1.0611x over previous
import functools

import jax
import jax.numpy as jnp
from jax import lax
from jax.experimental import pallas as pl
from jax.experimental.pallas import tpu as pltpu
from jax.experimental.pallas import tpu_sc as plsc

HIDDEN = 1024
FFN = 512
NUM_EXPERTS = 8
TOP_K = 2
T = 2048
P = T * TOP_K
M = 256
NB = 24
NR = NB * M
CHUNK = 512


def _route_body(x_ref, rw_ref, tw0_ref, tw1_ref, dk0_ref, dk1_ref,
                bexp_ref, nv_ref, oh0_s, oh1_s, r0_s, r1_s):
    x = x_ref[...]
    logits = jnp.dot(x, rw_ref[...], preferred_element_type=jnp.float32)
    probs = jax.nn.softmax(logits, axis=-1)
    lane = jax.lax.broadcasted_iota(jnp.int32, probs.shape, 1)
    m1 = jnp.max(probs, axis=-1, keepdims=True)
    i1 = jnp.min(jnp.where(probs == m1, lane, NUM_EXPERTS), axis=-1,
                 keepdims=True)
    oh0 = lane == i1
    masked = jnp.where(oh0, -jnp.inf, probs)
    m2 = jnp.max(masked, axis=-1, keepdims=True)
    i2 = jnp.min(jnp.where(masked == m2, lane, NUM_EXPERTS), axis=-1,
                 keepdims=True)
    oh1 = lane == i2
    s = m1 + m2
    ones_lg = jnp.ones((1, 128), jnp.float32)
    tw0_ref[...] = (m1 / s) * ones_lg
    tw1_ref[...] = (m2 / s) * ones_lg
    oh0_s[...] = oh0.astype(jnp.float32)
    oh1_s[...] = oh1.astype(jnp.float32)

    tri = (jax.lax.broadcasted_iota(jnp.int32, (CHUNK, CHUNK), 0)
           > jax.lax.broadcasted_iota(jnp.int32, (CHUNK, CHUNK), 1)
           ).astype(jnp.float32)

    zero8 = jnp.zeros((1, NUM_EXPERTS), jnp.float32)
    cnt0, cnt1 = zero8, zero8
    for c in range(T // CHUNK):
        o0 = oh0_s[pl.ds(c * CHUNK, CHUNK), :]
        o1 = oh1_s[pl.ds(c * CHUNK, CHUNK), :]
        r0_s[pl.ds(c * CHUNK, CHUNK), :] = (
            jnp.dot(tri, o0, preferred_element_type=jnp.float32) + cnt0)
        r1_s[pl.ds(c * CHUNK, CHUNK), :] = (
            jnp.dot(tri, o1, preferred_element_type=jnp.float32) + cnt1)
        cnt0 = cnt0 + jnp.sum(o0, axis=0, keepdims=True)
        cnt1 = cnt1 + jnp.sum(o1, axis=0, keepdims=True)
    cnt = cnt0 + cnt1
    nb = jnp.floor((cnt + (M - 1)) * (1.0 / M))
    pc = nb * M
    e_iota0 = jax.lax.broadcasted_iota(jnp.int32, (NUM_EXPERTS, NUM_EXPERTS), 0)
    e_iota1 = jax.lax.broadcasted_iota(jnp.int32, (NUM_EXPERTS, NUM_EXPERTS), 1)
    excl = (e_iota0 < e_iota1).astype(jnp.float32)
    incl = (e_iota0 <= e_iota1).astype(jnp.float32)
    padoff = jnp.dot(pc, excl, preferred_element_type=jnp.float32)
    cumnb = jnp.dot(nb, incl, preferred_element_type=jnp.float32)

    d0 = jnp.sum((padoff + r0_s[...]) * oh0_s[...], axis=-1, keepdims=True)
    d1 = jnp.sum((padoff + cnt0 + r1_s[...]) * oh1_s[...], axis=-1,
                 keepdims=True)
    dk0_ref[...] = d0.astype(jnp.int32)
    dk1_ref[...] = d1.astype(jnp.int32)

    b_iota = jax.lax.broadcasted_iota(jnp.int32, (1, NB), 1).astype(jnp.float32)
    b_iota = jnp.minimum(b_iota, cumnb[0, NUM_EXPERTS - 1] - 1.0)
    bexp = jnp.zeros((1, NB), jnp.float32)
    for e in range(NUM_EXPERTS):
        bexp = bexp + jnp.where(b_iota >= cumnb[0, e], 1.0, 0.0)
    bexp_ref[...] = jnp.minimum(bexp, NUM_EXPERTS - 1).astype(jnp.int32)
    nv_ref[...] = cumnb[0:1, NUM_EXPERTS - 1:NUM_EXPERTS].astype(jnp.int32)


def _route(x, rw):
    return pl.pallas_call(
        _route_body,
        in_specs=[
            pl.BlockSpec((T, HIDDEN), lambda: (0, 0)),
            pl.BlockSpec((HIDDEN, NUM_EXPERTS), lambda: (0, 0)),
        ],
        out_specs=[
            pl.BlockSpec((T, 128), lambda: (0, 0)),
            pl.BlockSpec((T, 128), lambda: (0, 0)),
            pl.BlockSpec((T, 1), lambda: (0, 0)),
            pl.BlockSpec((T, 1), lambda: (0, 0)),
            pl.BlockSpec((1, NB), lambda: (0, 0)),
            pl.BlockSpec((1, 1), lambda: (0, 0)),
        ],
        out_shape=[
            jax.ShapeDtypeStruct((T, 128), jnp.float32),
            jax.ShapeDtypeStruct((T, 128), jnp.float32),
            jax.ShapeDtypeStruct((T, 1), jnp.int32),
            jax.ShapeDtypeStruct((T, 1), jnp.int32),
            jax.ShapeDtypeStruct((1, NB), jnp.int32),
            jax.ShapeDtypeStruct((1, 1), jnp.int32),
        ],
        scratch_shapes=[pltpu.VMEM((T, NUM_EXPERTS), jnp.float32)] * 4,
    )(x, rw)


def _gmm_body(bexp_ref, nv_ref, xs_ref, ws_ref, g_ref, u_ref, d_ref, ys_ref):
    b = pl.program_id(0)

    @pl.when(b < nv_ref[0])
    def _():
        xb = xs_ref[...].astype(jnp.bfloat16)
        a = jnp.dot(xb, g_ref[0].astype(jnp.bfloat16),
                    preferred_element_type=jnp.float32)
        u = jnp.dot(xb, u_ref[0].astype(jnp.bfloat16),
                    preferred_element_type=jnp.float32)
        w = ws_ref[...][:, 0:1]
        h = (jax.nn.silu(a) * u * w).astype(jnp.bfloat16)
        ys_ref[...] = jnp.dot(h, d_ref[0].astype(jnp.bfloat16),
                              preferred_element_type=jnp.float32)


def _gmm(bexp, nv, xs, ws, gw, uw, dw):
    grid_spec = pltpu.PrefetchScalarGridSpec(
        num_scalar_prefetch=2,
        grid=(NB,),
        in_specs=[
            pl.BlockSpec((M, HIDDEN),
                         lambda b, be, nv: (jnp.where(b < nv[0], b, 0), 0)),
            pl.BlockSpec((M, 128),
                         lambda b, be, nv: (jnp.where(b < nv[0], b, 0), 0)),
            pl.BlockSpec((1, HIDDEN, FFN),
                         lambda b, be, nv: (be[b], 0, 0)),
            pl.BlockSpec((1, HIDDEN, FFN),
                         lambda b, be, nv: (be[b], 0, 0)),
            pl.BlockSpec((1, FFN, HIDDEN),
                         lambda b, be, nv: (be[b], 0, 0)),
        ],
        out_specs=pl.BlockSpec(
            (M, HIDDEN), lambda b, be, nv: (jnp.where(b < nv[0], b, NB), 0)),
    )
    return pl.pallas_call(
        _gmm_body,
        grid_spec=grid_spec,
        out_shape=jax.ShapeDtypeStruct((NR + M, HIDDEN), jnp.float32),
    )(bexp, nv, xs, ws, gw, uw, dw)


NW = 32
TOK_W = T // NW
SCCH = 32
LG = 16


@functools.cache
def _sc_kernels():
    mesh = plsc.VectorSubcoreMesh(core_axis_name="c", subcore_axis_name="s",
                                  num_cores=2, num_subcores=16)

    @functools.partial(
        pl.kernel,
        out_type=[
            jax.ShapeDtypeStruct((NR, HIDDEN), jnp.float32),
            jax.ShapeDtypeStruct((NR, 128), jnp.float32),
        ],
        mesh=mesh,
        scratch_types=[
            pltpu.VMEM((TOK_W, HIDDEN), jnp.float32),
            pltpu.VMEM((TOK_W,), jnp.int32),
            pltpu.VMEM((TOK_W,), jnp.int32),
            pltpu.VMEM((TOK_W, 128), jnp.float32),
            pltpu.VMEM((TOK_W, 128), jnp.float32),
            pltpu.SemaphoreType.DMA,
            pltpu.SemaphoreType.DMA,
            pltpu.SemaphoreType.DMA,
            pltpu.SemaphoreType.DMA,
            pltpu.SemaphoreType.DMA,
        ],
    )
    def _scatter_k(x_hbm, dk0_hbm, dk1_hbm, tw0_hbm, tw1_hbm, xs_hbm, ws_hbm,
                   xv, i0v, i1v, w0v, w1v, s0, s1, s2, s3, s4):
        wid = lax.axis_index("s") * 2 + lax.axis_index("c")
        base = wid * TOK_W
        cps = [
            pltpu.async_copy(x_hbm.at[pl.ds(base, TOK_W)], xv, s0),
            pltpu.async_copy(dk0_hbm.at[pl.ds(base, TOK_W)], i0v, s1),
            pltpu.async_copy(dk1_hbm.at[pl.ds(base, TOK_W)], i1v, s2),
            pltpu.async_copy(tw0_hbm.at[pl.ds(base, TOK_W)], w0v, s3),
            pltpu.async_copy(tw1_hbm.at[pl.ds(base, TOK_W)], w1v, s4),
        ]
        for cp in cps:
            cp.wait()
        cps = [
            pltpu.async_copy(xv, xs_hbm.at[i0v], s0),
            pltpu.async_copy(xv, xs_hbm.at[i1v], s1),
            pltpu.async_copy(w0v, ws_hbm.at[i0v], s2),
            pltpu.async_copy(w1v, ws_hbm.at[i1v], s3),
        ]
        for cp in cps:
            cp.wait()

    CCH = 16
    NCH = TOK_W // CCH

    @functools.partial(
        pl.kernel,
        out_type=jax.ShapeDtypeStruct((T, HIDDEN), jnp.float32),
        mesh=mesh,
        scratch_types=[
            pltpu.VMEM((2, CCH, HIDDEN), jnp.float32),
            pltpu.VMEM((2, CCH, HIDDEN), jnp.float32),
            pltpu.VMEM((TOK_W,), jnp.int32),
            pltpu.VMEM((TOK_W,), jnp.int32),
            pltpu.SemaphoreType.DMA,
            pltpu.SemaphoreType.DMA,
            pltpu.SemaphoreType.DMA,
            pltpu.SemaphoreType.DMA,
            pltpu.SemaphoreType.DMA,
        ],
    )
    def _combine_k(ys_hbm, dk0_hbm, dk1_hbm, out_hbm,
                   ya, yb, i0v, i1v, sa0, sa1, sb0, sb1, sw):
        wid = lax.axis_index("s") * 2 + lax.axis_index("c")
        base = wid * TOK_W
        cp0 = pltpu.async_copy(dk0_hbm.at[pl.ds(base, TOK_W)], i0v, sa0)
        cp1 = pltpu.async_copy(dk1_hbm.at[pl.ds(base, TOK_W)], i1v, sb0)
        cp0.wait()
        cp1.wait()
        sa = [sa0, sa1]
        sb = [sb0, sb1]

        def gather(c):
            buf = c % 2
            ca = pltpu.async_copy(
                ys_hbm.at[i0v.at[pl.ds(c * CCH, CCH)]], ya.at[buf], sa[buf])
            cb = pltpu.async_copy(
                ys_hbm.at[i1v.at[pl.ds(c * CCH, CCH)]], yb.at[buf], sb[buf])
            return ca, cb

        pend = gather(0)
        wr = None
        for c in range(NCH):
            buf = c % 2
            ca, cb = pend
            if wr is not None:
                wr.wait()
            if c + 1 < NCH:
                pend = gather(c + 1)
            ca.wait()
            cb.wait()

            @plsc.parallel_loop(0, CCH * (HIDDEN // LG), 1, unroll=4)
            def _add(g):
                i = g // (HIDDEN // LG)
                j = g % (HIDDEN // LG)
                ya[buf, i, pl.ds(j * LG, LG)] = (
                    ya[buf, i, pl.ds(j * LG, LG)]
                    + yb[buf, i, pl.ds(j * LG, LG)])
            wr = pltpu.async_copy(
                ya.at[buf], out_hbm.at[pl.ds(base + c * CCH, CCH)], sw)
        wr.wait()

    return _scatter_k, _combine_k


def kernel(hidden_states, router_w, gate_w, up_w, down_w):
    B, S, H = hidden_states.shape
    scatter_k, combine_k = _sc_kernels()
    x = hidden_states.reshape(-1, H)
    tw0, tw1, dk0, dk1, bexp, nv = _route(x, router_w)
    dk0 = dk0.reshape(-1)
    dk1 = dk1.reshape(-1)
    xs, ws = scatter_k(x, dk0, dk1, tw0, tw1)
    ys = _gmm(bexp.reshape(-1), nv.reshape(-1), xs, ws,
              gate_w, up_w, down_w)
    out = combine_k(ys, dk0, dk1)
    return out.reshape(B, S, H)


# speedup vs baseline: 1.1326x; 1.0674x over previous
import functools

import jax
import jax.numpy as jnp
from jax import lax
from jax.experimental import pallas as pl
from jax.experimental.pallas import tpu as pltpu
from jax.experimental.pallas import tpu_sc as plsc

HIDDEN = 1024
FFN = 512
NUM_EXPERTS = 8
TOP_K = 2
T = 2048
P = T * TOP_K
M = 512
NB = 15
NR = NB * M
CHUNK = 512


def _route_body(x_ref, rw_ref, tw0_ref, tw1_ref, dk0_ref, dk1_ref,
                bexp_ref, nv_ref, oh0_s, oh1_s, r0_s, r1_s):
    x = x_ref[...]
    logits = jnp.dot(x, rw_ref[...], preferred_element_type=jnp.float32)
    probs = jax.nn.softmax(logits, axis=-1)
    lane = jax.lax.broadcasted_iota(jnp.int32, probs.shape, 1)
    m1 = jnp.max(probs, axis=-1, keepdims=True)
    i1 = jnp.min(jnp.where(probs == m1, lane, NUM_EXPERTS), axis=-1,
                 keepdims=True)
    oh0 = lane == i1
    masked = jnp.where(oh0, -jnp.inf, probs)
    m2 = jnp.max(masked, axis=-1, keepdims=True)
    i2 = jnp.min(jnp.where(masked == m2, lane, NUM_EXPERTS), axis=-1,
                 keepdims=True)
    oh1 = lane == i2
    s = m1 + m2
    ones_lg = jnp.ones((1, 128), jnp.float32)
    tw0_ref[...] = (m1 / s) * ones_lg
    tw1_ref[...] = (m2 / s) * ones_lg
    oh0_s[...] = oh0.astype(jnp.float32)
    oh1_s[...] = oh1.astype(jnp.float32)

    tri = (jax.lax.broadcasted_iota(jnp.int32, (CHUNK, CHUNK), 0)
           > jax.lax.broadcasted_iota(jnp.int32, (CHUNK, CHUNK), 1)
           ).astype(jnp.float32)

    zero8 = jnp.zeros((1, NUM_EXPERTS), jnp.float32)
    cnt0, cnt1 = zero8, zero8
    for c in range(T // CHUNK):
        o0 = oh0_s[pl.ds(c * CHUNK, CHUNK), :]
        o1 = oh1_s[pl.ds(c * CHUNK, CHUNK), :]
        r0_s[pl.ds(c * CHUNK, CHUNK), :] = (
            jnp.dot(tri, o0, preferred_element_type=jnp.float32) + cnt0)
        r1_s[pl.ds(c * CHUNK, CHUNK), :] = (
            jnp.dot(tri, o1, preferred_element_type=jnp.float32) + cnt1)
        cnt0 = cnt0 + jnp.sum(o0, axis=0, keepdims=True)
        cnt1 = cnt1 + jnp.sum(o1, axis=0, keepdims=True)
    cnt = cnt0 + cnt1
    nb = jnp.floor((cnt + (M - 1)) * (1.0 / M))
    pc = nb * M
    e_iota0 = jax.lax.broadcasted_iota(jnp.int32, (NUM_EXPERTS, NUM_EXPERTS), 0)
    e_iota1 = jax.lax.broadcasted_iota(jnp.int32, (NUM_EXPERTS, NUM_EXPERTS), 1)
    excl = (e_iota0 < e_iota1).astype(jnp.float32)
    incl = (e_iota0 <= e_iota1).astype(jnp.float32)
    padoff = jnp.dot(pc, excl, preferred_element_type=jnp.float32)
    cumnb = jnp.dot(nb, incl, preferred_element_type=jnp.float32)

    d0 = jnp.sum((padoff + r0_s[...]) * oh0_s[...], axis=-1, keepdims=True)
    d1 = jnp.sum((padoff + cnt0 + r1_s[...]) * oh1_s[...], axis=-1,
                 keepdims=True)
    dk0_ref[...] = d0.astype(jnp.int32)
    dk1_ref[...] = d1.astype(jnp.int32)

    b_iota = jax.lax.broadcasted_iota(jnp.int32, (1, NB), 1).astype(jnp.float32)
    b_iota = jnp.minimum(b_iota, cumnb[0, NUM_EXPERTS - 1] - 1.0)
    bexp = jnp.zeros((1, NB), jnp.float32)
    for e in range(NUM_EXPERTS):
        bexp = bexp + jnp.where(b_iota >= cumnb[0, e], 1.0, 0.0)
    bexp_ref[...] = jnp.minimum(bexp, NUM_EXPERTS - 1).astype(jnp.int32)
    nv_ref[...] = cumnb[0:1, NUM_EXPERTS - 1:NUM_EXPERTS].astype(jnp.int32)


def _route(x, rw):
    return pl.pallas_call(
        _route_body,
        in_specs=[
            pl.BlockSpec((T, HIDDEN), lambda: (0, 0)),
            pl.BlockSpec((HIDDEN, NUM_EXPERTS), lambda: (0, 0)),
        ],
        out_specs=[
            pl.BlockSpec((T, 128), lambda: (0, 0)),
            pl.BlockSpec((T, 128), lambda: (0, 0)),
            pl.BlockSpec((T, 1), lambda: (0, 0)),
            pl.BlockSpec((T, 1), lambda: (0, 0)),
            pl.BlockSpec((1, NB), lambda: (0, 0)),
            pl.BlockSpec((1, 1), lambda: (0, 0)),
        ],
        out_shape=[
            jax.ShapeDtypeStruct((T, 128), jnp.float32),
            jax.ShapeDtypeStruct((T, 128), jnp.float32),
            jax.ShapeDtypeStruct((T, 1), jnp.int32),
            jax.ShapeDtypeStruct((T, 1), jnp.int32),
            jax.ShapeDtypeStruct((1, NB), jnp.int32),
            jax.ShapeDtypeStruct((1, 1), jnp.int32),
        ],
        scratch_shapes=[pltpu.VMEM((T, NUM_EXPERTS), jnp.float32)] * 4,
    )(x, rw)


def _gmm_body(bexp_ref, nv_ref, xs_ref, ws_ref, g_ref, u_ref, d_ref, ys_ref):
    b = pl.program_id(0)

    @pl.when(b < nv_ref[0])
    def _():
        xb = xs_ref[...].astype(jnp.bfloat16)
        a = jnp.dot(xb, g_ref[0].astype(jnp.bfloat16),
                    preferred_element_type=jnp.float32)
        u = jnp.dot(xb, u_ref[0].astype(jnp.bfloat16),
                    preferred_element_type=jnp.float32)
        w = ws_ref[...][:, 0:1]
        h = (jax.nn.silu(a) * u * w).astype(jnp.bfloat16)
        ys_ref[...] = jnp.dot(h, d_ref[0].astype(jnp.bfloat16),
                              preferred_element_type=jnp.float32)


def _gmm(bexp, nv, xs, ws, gw, uw, dw):
    grid_spec = pltpu.PrefetchScalarGridSpec(
        num_scalar_prefetch=2,
        grid=(NB,),
        in_specs=[
            pl.BlockSpec((M, HIDDEN),
                         lambda b, be, nv: (jnp.where(b < nv[0], b, 0), 0)),
            pl.BlockSpec((M, 128),
                         lambda b, be, nv: (jnp.where(b < nv[0], b, 0), 0)),
            pl.BlockSpec((1, HIDDEN, FFN),
                         lambda b, be, nv: (be[b], 0, 0)),
            pl.BlockSpec((1, HIDDEN, FFN),
                         lambda b, be, nv: (be[b], 0, 0)),
            pl.BlockSpec((1, FFN, HIDDEN),
                         lambda b, be, nv: (be[b], 0, 0)),
        ],
        out_specs=pl.BlockSpec(
            (M, HIDDEN), lambda b, be, nv: (jnp.where(b < nv[0], b, NB), 0)),
    )
    return pl.pallas_call(
        _gmm_body,
        grid_spec=grid_spec,
        out_shape=jax.ShapeDtypeStruct((NR + M, HIDDEN), jnp.float32),
    )(bexp, nv, xs, ws, gw, uw, dw)


NW = 32
TOK_W = T // NW
SCCH = 32
LG = 16


@functools.cache
def _sc_kernels():
    mesh = plsc.VectorSubcoreMesh(core_axis_name="c", subcore_axis_name="s",
                                  num_cores=2, num_subcores=16)

    @functools.partial(
        pl.kernel,
        out_type=[
            jax.ShapeDtypeStruct((NR, HIDDEN), jnp.float32),
            jax.ShapeDtypeStruct((NR, 128), jnp.float32),
        ],
        mesh=mesh,
        scratch_types=[
            pltpu.VMEM((TOK_W, HIDDEN), jnp.float32),
            pltpu.VMEM((TOK_W,), jnp.int32),
            pltpu.VMEM((TOK_W,), jnp.int32),
            pltpu.VMEM((TOK_W, 128), jnp.float32),
            pltpu.VMEM((TOK_W, 128), jnp.float32),
            pltpu.SemaphoreType.DMA,
            pltpu.SemaphoreType.DMA,
            pltpu.SemaphoreType.DMA,
            pltpu.SemaphoreType.DMA,
            pltpu.SemaphoreType.DMA,
        ],
    )
    def _scatter_k(x_hbm, dk0_hbm, dk1_hbm, tw0_hbm, tw1_hbm, xs_hbm, ws_hbm,
                   xv, i0v, i1v, w0v, w1v, s0, s1, s2, s3, s4):
        wid = lax.axis_index("s") * 2 + lax.axis_index("c")
        base = wid * TOK_W
        cps = [
            pltpu.async_copy(x_hbm.at[pl.ds(base, TOK_W)], xv, s0),
            pltpu.async_copy(dk0_hbm.at[pl.ds(base, TOK_W)], i0v, s1),
            pltpu.async_copy(dk1_hbm.at[pl.ds(base, TOK_W)], i1v, s2),
            pltpu.async_copy(tw0_hbm.at[pl.ds(base, TOK_W)], w0v, s3),
            pltpu.async_copy(tw1_hbm.at[pl.ds(base, TOK_W)], w1v, s4),
        ]
        for cp in cps:
            cp.wait()
        cps = [
            pltpu.async_copy(xv, xs_hbm.at[i0v], s0),
            pltpu.async_copy(xv, xs_hbm.at[i1v], s1),
            pltpu.async_copy(w0v, ws_hbm.at[i0v], s2),
            pltpu.async_copy(w1v, ws_hbm.at[i1v], s3),
        ]
        for cp in cps:
            cp.wait()

    CCH = 16
    NCH = TOK_W // CCH

    @functools.partial(
        pl.kernel,
        out_type=jax.ShapeDtypeStruct((T, HIDDEN), jnp.float32),
        mesh=mesh,
        scratch_types=[
            pltpu.VMEM((2, CCH, HIDDEN), jnp.float32),
            pltpu.VMEM((2, CCH, HIDDEN), jnp.float32),
            pltpu.VMEM((TOK_W,), jnp.int32),
            pltpu.VMEM((TOK_W,), jnp.int32),
            pltpu.SemaphoreType.DMA,
            pltpu.SemaphoreType.DMA,
            pltpu.SemaphoreType.DMA,
            pltpu.SemaphoreType.DMA,
            pltpu.SemaphoreType.DMA,
        ],
    )
    def _combine_k(ys_hbm, dk0_hbm, dk1_hbm, out_hbm,
                   ya, yb, i0v, i1v, sa0, sa1, sb0, sb1, sw):
        wid = lax.axis_index("s") * 2 + lax.axis_index("c")
        base = wid * TOK_W
        cp0 = pltpu.async_copy(dk0_hbm.at[pl.ds(base, TOK_W)], i0v, sa0)
        cp1 = pltpu.async_copy(dk1_hbm.at[pl.ds(base, TOK_W)], i1v, sb0)
        cp0.wait()
        cp1.wait()
        sa = [sa0, sa1]
        sb = [sb0, sb1]

        def gather(c):
            buf = c % 2
            ca = pltpu.async_copy(
                ys_hbm.at[i0v.at[pl.ds(c * CCH, CCH)]], ya.at[buf], sa[buf])
            cb = pltpu.async_copy(
                ys_hbm.at[i1v.at[pl.ds(c * CCH, CCH)]], yb.at[buf], sb[buf])
            return ca, cb

        pend = gather(0)
        wr = None
        for c in range(NCH):
            buf = c % 2
            ca, cb = pend
            if wr is not None:
                wr.wait()
            if c + 1 < NCH:
                pend = gather(c + 1)
            ca.wait()
            cb.wait()

            @plsc.parallel_loop(0, CCH * (HIDDEN // LG), 1, unroll=4)
            def _add(g):
                i = g // (HIDDEN // LG)
                j = g % (HIDDEN // LG)
                ya[buf, i, pl.ds(j * LG, LG)] = (
                    ya[buf, i, pl.ds(j * LG, LG)]
                    + yb[buf, i, pl.ds(j * LG, LG)])
            wr = pltpu.async_copy(
                ya.at[buf], out_hbm.at[pl.ds(base + c * CCH, CCH)], sw)
        wr.wait()

    return _scatter_k, _combine_k


def kernel(hidden_states, router_w, gate_w, up_w, down_w):
    B, S, H = hidden_states.shape
    scatter_k, combine_k = _sc_kernels()
    x = hidden_states.reshape(-1, H)
    tw0, tw1, dk0, dk1, bexp, nv = _route(x, router_w)
    dk0 = dk0.reshape(-1)
    dk1 = dk1.reshape(-1)
    xs, ws = scatter_k(x, dk0, dk1, tw0, tw1)
    ys = _gmm(bexp.reshape(-1), nv.reshape(-1), xs, ws,
              gate_w, up_w, down_w)
    out = combine_k(ys, dk0, dk1)
    return out.reshape(B, S, H)


# speedup vs baseline: 1.1382x; 1.0049x over previous
import functools

import jax
import jax.numpy as jnp
from jax import lax
from jax.experimental import pallas as pl
from jax.experimental.pallas import tpu as pltpu
from jax.experimental.pallas import tpu_sc as plsc

HIDDEN = 1024
FFN = 512
NUM_EXPERTS = 8
TOP_K = 2
T = 2048
P = T * TOP_K
M = 512
NB = 15
NR = NB * M
CHUNK = 512


def _route_body(x_ref, rw_ref, tw0_ref, tw1_ref, dk0_ref, dk1_ref,
                bexp_ref, nv_ref, oh0_s, oh1_s, r0_s, r1_s):
    x = x_ref[...]
    logits = jnp.dot(x, rw_ref[...], preferred_element_type=jnp.float32)
    lane = jax.lax.broadcasted_iota(jnp.int32, logits.shape, 1)
    m1 = jnp.max(logits, axis=-1, keepdims=True)
    i1 = jnp.min(jnp.where(logits == m1, lane, NUM_EXPERTS), axis=-1,
                 keepdims=True)
    oh0 = lane == i1
    masked = jnp.where(oh0, -jnp.inf, logits)
    m2 = jnp.max(masked, axis=-1, keepdims=True)
    i2 = jnp.min(jnp.where(masked == m2, lane, NUM_EXPERTS), axis=-1,
                 keepdims=True)
    oh1 = lane == i2
    e1 = jnp.exp(m1 - m1)
    e2 = jnp.exp(m2 - m1)
    s = e1 + e2
    ones_lg = jnp.ones((1, 128), jnp.float32)
    tw0_ref[...] = (e1 / s) * ones_lg
    tw1_ref[...] = (e2 / s) * ones_lg
    oh0_s[...] = oh0.astype(jnp.float32)
    oh1_s[...] = oh1.astype(jnp.float32)

    tri = (jax.lax.broadcasted_iota(jnp.int32, (CHUNK, CHUNK), 0)
           > jax.lax.broadcasted_iota(jnp.int32, (CHUNK, CHUNK), 1)
           ).astype(jnp.float32)

    zero8 = jnp.zeros((1, NUM_EXPERTS), jnp.float32)
    cnt0, cnt1 = zero8, zero8
    for c in range(T // CHUNK):
        o0 = oh0_s[pl.ds(c * CHUNK, CHUNK), :]
        o1 = oh1_s[pl.ds(c * CHUNK, CHUNK), :]
        r0_s[pl.ds(c * CHUNK, CHUNK), :] = (
            jnp.dot(tri, o0, preferred_element_type=jnp.float32) + cnt0)
        r1_s[pl.ds(c * CHUNK, CHUNK), :] = (
            jnp.dot(tri, o1, preferred_element_type=jnp.float32) + cnt1)
        cnt0 = cnt0 + jnp.sum(o0, axis=0, keepdims=True)
        cnt1 = cnt1 + jnp.sum(o1, axis=0, keepdims=True)
    cnt = cnt0 + cnt1
    nb = jnp.floor((cnt + (M - 1)) * (1.0 / M))
    pc = nb * M
    e_iota0 = jax.lax.broadcasted_iota(jnp.int32, (NUM_EXPERTS, NUM_EXPERTS), 0)
    e_iota1 = jax.lax.broadcasted_iota(jnp.int32, (NUM_EXPERTS, NUM_EXPERTS), 1)
    excl = (e_iota0 < e_iota1).astype(jnp.float32)
    incl = (e_iota0 <= e_iota1).astype(jnp.float32)
    padoff = jnp.dot(pc, excl, preferred_element_type=jnp.float32)
    cumnb = jnp.dot(nb, incl, preferred_element_type=jnp.float32)

    d0 = jnp.sum((padoff + r0_s[...]) * oh0_s[...], axis=-1, keepdims=True)
    d1 = jnp.sum((padoff + cnt0 + r1_s[...]) * oh1_s[...], axis=-1,
                 keepdims=True)
    dk0_ref[...] = d0.astype(jnp.int32)
    dk1_ref[...] = d1.astype(jnp.int32)

    b_iota = jax.lax.broadcasted_iota(jnp.int32, (1, NB), 1).astype(jnp.float32)
    b_iota = jnp.minimum(b_iota, cumnb[0, NUM_EXPERTS - 1] - 1.0)
    bexp = jnp.zeros((1, NB), jnp.float32)
    for e in range(NUM_EXPERTS):
        bexp = bexp + jnp.where(b_iota >= cumnb[0, e], 1.0, 0.0)
    bexp_ref[...] = jnp.minimum(bexp, NUM_EXPERTS - 1).astype(jnp.int32)
    nv_ref[...] = cumnb[0:1, NUM_EXPERTS - 1:NUM_EXPERTS].astype(jnp.int32)


def _route(x, rw):
    return pl.pallas_call(
        _route_body,
        in_specs=[
            pl.BlockSpec((T, HIDDEN), lambda: (0, 0)),
            pl.BlockSpec((HIDDEN, NUM_EXPERTS), lambda: (0, 0)),
        ],
        out_specs=[
            pl.BlockSpec((T, 128), lambda: (0, 0)),
            pl.BlockSpec((T, 128), lambda: (0, 0)),
            pl.BlockSpec((T, 1), lambda: (0, 0)),
            pl.BlockSpec((T, 1), lambda: (0, 0)),
            pl.BlockSpec((1, NB), lambda: (0, 0)),
            pl.BlockSpec((1, 1), lambda: (0, 0)),
        ],
        out_shape=[
            jax.ShapeDtypeStruct((T, 128), jnp.float32),
            jax.ShapeDtypeStruct((T, 128), jnp.float32),
            jax.ShapeDtypeStruct((T, 1), jnp.int32),
            jax.ShapeDtypeStruct((T, 1), jnp.int32),
            jax.ShapeDtypeStruct((1, NB), jnp.int32),
            jax.ShapeDtypeStruct((1, 1), jnp.int32),
        ],
        scratch_shapes=[pltpu.VMEM((T, NUM_EXPERTS), jnp.float32)] * 4,
    )(x, rw)


def _gmm_body(bexp_ref, nv_ref, xs_ref, ws_ref, g_ref, u_ref, d_ref, ys_ref):
    b = pl.program_id(0)

    @pl.when(b < nv_ref[0])
    def _():
        xb = xs_ref[...].astype(jnp.bfloat16)
        a = jnp.dot(xb, g_ref[0].astype(jnp.bfloat16),
                    preferred_element_type=jnp.float32)
        u = jnp.dot(xb, u_ref[0].astype(jnp.bfloat16),
                    preferred_element_type=jnp.float32)
        w = ws_ref[...][:, 0:1]
        h = (jax.nn.silu(a) * u * w).astype(jnp.bfloat16)
        ys_ref[...] = jnp.dot(h, d_ref[0].astype(jnp.bfloat16),
                              preferred_element_type=jnp.float32)


def _gmm(bexp, nv, xs, ws, gw, uw, dw):
    grid_spec = pltpu.PrefetchScalarGridSpec(
        num_scalar_prefetch=2,
        grid=(NB,),
        in_specs=[
            pl.BlockSpec((M, HIDDEN),
                         lambda b, be, nv: (jnp.where(b < nv[0], b, 0), 0)),
            pl.BlockSpec((M, 128),
                         lambda b, be, nv: (jnp.where(b < nv[0], b, 0), 0)),
            pl.BlockSpec((1, HIDDEN, FFN),
                         lambda b, be, nv: (be[b], 0, 0)),
            pl.BlockSpec((1, HIDDEN, FFN),
                         lambda b, be, nv: (be[b], 0, 0)),
            pl.BlockSpec((1, FFN, HIDDEN),
                         lambda b, be, nv: (be[b], 0, 0)),
        ],
        out_specs=pl.BlockSpec(
            (M, HIDDEN), lambda b, be, nv: (jnp.where(b < nv[0], b, NB), 0)),
    )
    return pl.pallas_call(
        _gmm_body,
        grid_spec=grid_spec,
        out_shape=jax.ShapeDtypeStruct((NR + M, HIDDEN), jnp.float32),
    )(bexp, nv, xs, ws, gw, uw, dw)


NW = 32
TOK_W = T // NW
SCCH = 32
LG = 16


@functools.cache
def _sc_kernels():
    mesh = plsc.VectorSubcoreMesh(core_axis_name="c", subcore_axis_name="s",
                                  num_cores=2, num_subcores=16)

    @functools.partial(
        pl.kernel,
        out_type=[
            jax.ShapeDtypeStruct((NR, HIDDEN), jnp.float32),
            jax.ShapeDtypeStruct((NR, 128), jnp.float32),
        ],
        mesh=mesh,
        scratch_types=[
            pltpu.VMEM((TOK_W, HIDDEN), jnp.float32),
            pltpu.VMEM((TOK_W,), jnp.int32),
            pltpu.VMEM((TOK_W,), jnp.int32),
            pltpu.VMEM((TOK_W, 128), jnp.float32),
            pltpu.VMEM((TOK_W, 128), jnp.float32),
            pltpu.SemaphoreType.DMA,
            pltpu.SemaphoreType.DMA,
            pltpu.SemaphoreType.DMA,
            pltpu.SemaphoreType.DMA,
            pltpu.SemaphoreType.DMA,
        ],
    )
    def _scatter_k(x_hbm, dk0_hbm, dk1_hbm, tw0_hbm, tw1_hbm, xs_hbm, ws_hbm,
                   xv, i0v, i1v, w0v, w1v, s0, s1, s2, s3, s4):
        wid = lax.axis_index("s") * 2 + lax.axis_index("c")
        base = wid * TOK_W
        cps = [
            pltpu.async_copy(x_hbm.at[pl.ds(base, TOK_W)], xv, s0),
            pltpu.async_copy(dk0_hbm.at[pl.ds(base, TOK_W)], i0v, s1),
            pltpu.async_copy(dk1_hbm.at[pl.ds(base, TOK_W)], i1v, s2),
            pltpu.async_copy(tw0_hbm.at[pl.ds(base, TOK_W)], w0v, s3),
            pltpu.async_copy(tw1_hbm.at[pl.ds(base, TOK_W)], w1v, s4),
        ]
        for cp in cps:
            cp.wait()
        cps = [
            pltpu.async_copy(xv, xs_hbm.at[i0v], s0),
            pltpu.async_copy(xv, xs_hbm.at[i1v], s1),
            pltpu.async_copy(w0v, ws_hbm.at[i0v], s2),
            pltpu.async_copy(w1v, ws_hbm.at[i1v], s3),
        ]
        for cp in cps:
            cp.wait()

    CCH = 16
    NCH = TOK_W // CCH

    @functools.partial(
        pl.kernel,
        out_type=jax.ShapeDtypeStruct((T, HIDDEN), jnp.float32),
        mesh=mesh,
        scratch_types=[
            pltpu.VMEM((2, CCH, HIDDEN), jnp.float32),
            pltpu.VMEM((2, CCH, HIDDEN), jnp.float32),
            pltpu.VMEM((TOK_W,), jnp.int32),
            pltpu.VMEM((TOK_W,), jnp.int32),
            pltpu.SemaphoreType.DMA,
            pltpu.SemaphoreType.DMA,
            pltpu.SemaphoreType.DMA,
            pltpu.SemaphoreType.DMA,
            pltpu.SemaphoreType.DMA,
        ],
    )
    def _combine_k(ys_hbm, dk0_hbm, dk1_hbm, out_hbm,
                   ya, yb, i0v, i1v, sa0, sa1, sb0, sb1, sw):
        wid = lax.axis_index("s") * 2 + lax.axis_index("c")
        base = wid * TOK_W
        cp0 = pltpu.async_copy(dk0_hbm.at[pl.ds(base, TOK_W)], i0v, sa0)
        cp1 = pltpu.async_copy(dk1_hbm.at[pl.ds(base, TOK_W)], i1v, sb0)
        cp0.wait()
        cp1.wait()
        sa = [sa0, sa1]
        sb = [sb0, sb1]

        def gather(c):
            buf = c % 2
            ca = pltpu.async_copy(
                ys_hbm.at[i0v.at[pl.ds(c * CCH, CCH)]], ya.at[buf], sa[buf])
            cb = pltpu.async_copy(
                ys_hbm.at[i1v.at[pl.ds(c * CCH, CCH)]], yb.at[buf], sb[buf])
            return ca, cb

        pend = gather(0)
        wr = None
        for c in range(NCH):
            buf = c % 2
            ca, cb = pend
            if wr is not None:
                wr.wait()
            if c + 1 < NCH:
                pend = gather(c + 1)
            ca.wait()
            cb.wait()

            @plsc.parallel_loop(0, CCH * (HIDDEN // LG), 1, unroll=8)
            def _add(g):
                i = g // (HIDDEN // LG)
                j = g % (HIDDEN // LG)
                ya[buf, i, pl.ds(j * LG, LG)] = (
                    ya[buf, i, pl.ds(j * LG, LG)]
                    + yb[buf, i, pl.ds(j * LG, LG)])
            wr = pltpu.async_copy(
                ya.at[buf], out_hbm.at[pl.ds(base + c * CCH, CCH)], sw)
        wr.wait()

    return _scatter_k, _combine_k


def kernel(hidden_states, router_w, gate_w, up_w, down_w):
    B, S, H = hidden_states.shape
    scatter_k, combine_k = _sc_kernels()
    x = hidden_states.reshape(-1, H)
    tw0, tw1, dk0, dk1, bexp, nv = _route(x, router_w)
    dk0 = dk0.reshape(-1)
    dk1 = dk1.reshape(-1)
    xs, ws = scatter_k(x, dk0, dk1, tw0, tw1)
    ys = _gmm(bexp.reshape(-1), nv.reshape(-1), xs, ws,
              gate_w, up_w, down_w)
    out = combine_k(ys, dk0, dk1)
    return out.reshape(B, S, H)
